# baseline (speedup 1.0000x reference)
import functools

import jax
import jax.numpy as jnp
from jax import lax
from jax.experimental import pallas as pl
from jax.experimental.pallas import tpu as pltpu
from jax.experimental.pallas import tpu_sc as plsc

T = 2048
H = 2048
FF = 5632
E = 8
K = 2
C = 640
SLOTS = E * C
SENT = SLOTS
AUX_COEF = 0.001
Z_COEF = 0.001

FB = 512
NFB = FF // FB

NC = 2
NS = 16
NW = NC * NS


def _cumsum0(a):
    n, e = a.shape
    s = 1
    while s < n:
        a = a + jnp.concatenate([jnp.zeros((s, e), a.dtype), a[:-s]], axis=0)
        s *= 2
    return a


def _router_body(x_ref, wr_ref, dest_ref, wts_ref, aux_ref):
    x = x_ref[...]
    wr = wr_ref[...]
    logits = jnp.dot(x, wr, preferred_element_type=jnp.float32)
    m = jnp.max(logits, axis=-1, keepdims=True)
    ex = jnp.exp(logits - m)
    se = jnp.sum(ex, axis=-1, keepdims=True)
    probs = ex / se
    lse = m + jnp.log(se)

    lane = jax.lax.broadcasted_iota(jnp.int32, (T, E), 1)
    m0 = jnp.max(probs, axis=-1, keepdims=True)
    i0 = jnp.min(jnp.where(probs == m0, lane, E), axis=-1, keepdims=True)
    oh0 = lane == i0
    probs1 = jnp.where(oh0, -jnp.inf, probs)
    m1 = jnp.max(probs1, axis=-1, keepdims=True)
    i1 = jnp.min(jnp.where(probs1 == m1, lane, E), axis=-1, keepdims=True)
    oh1 = lane == i1

    oh0f = oh0.astype(jnp.float32)
    oh1f = oh1.astype(jnp.float32)
    cs0 = _cumsum0(oh0f)
    cnt0 = cs0[T - 1:T, :]
    pos0 = cs0 - oh0f
    cs1 = _cumsum0(oh1f)
    cnt1 = cs1[T - 1:T, :]
    pos1 = cs1 - oh1f + cnt0

    p0 = jnp.sum(pos0 * oh0f, axis=-1, keepdims=True).astype(jnp.int32)
    p1 = jnp.sum(pos1 * oh1f, axis=-1, keepdims=True).astype(jnp.int32)
    keep0 = p0 < C
    keep1 = p1 < C
    sw = m0 + m1
    w0 = jnp.where(keep0, m0 / sw, 0.0)
    w1 = jnp.where(keep1, m1 / sw, 0.0)
    d0 = jnp.where(keep0, i0 * C + p0, SENT)
    d1 = jnp.where(keep1, i1 * C + p1, SENT)

    col = jax.lax.broadcasted_iota(jnp.int32, (T, 8), 1)
    dest_ref[...] = jnp.where(col == 0, d0, jnp.where(col == 1, d1, 0))
    wts_ref[...] = jnp.where(col == 0, w0, jnp.where(col == 1, w1, 0.0))

    f = (cnt0 + cnt1) / T
    p_mean = jnp.sum(probs, axis=0, keepdims=True) / T
    aux = (AUX_COEF * E * jnp.sum(f * p_mean)
           + Z_COEF * jnp.sum(lse * lse) / T)
    aux_ref[...] = jnp.full((1, 1), aux, jnp.float32)


def _router(x, w_router):
    return pl.pallas_call(
        _router_body,
        out_shape=[
            jax.ShapeDtypeStruct((T, 8), jnp.int32),
            jax.ShapeDtypeStruct((T, 8), jnp.float32),
            jax.ShapeDtypeStruct((1, 1), jnp.float32),
        ],
    )(x, w_router)


def _src_build(dest2):
    mesh = plsc.VectorSubcoreMesh(core_axis_name="c", subcore_axis_name="s")

    @functools.partial(
        pl.kernel,
        out_type=jax.ShapeDtypeStruct((SLOTS,), jnp.int32),
        mesh=mesh,
        scratch_types=[
            pltpu.VMEM((2, T), jnp.int32),
            pltpu.VMEM((SLOTS,), jnp.int32),
        ],
        compiler_params=pltpu.CompilerParams(needs_layout_passes=False),
    )
    def k(dest_hbm, src_hbm, d_v, s_v):
        wid = lax.axis_index("s") * NC + lax.axis_index("c")

        @pl.when(wid == 0)
        def _():
            pltpu.sync_copy(dest_hbm, d_v)

            def init(b, carry):
                s_v[pl.ds(b * 16, 16)] = jnp.zeros((16,), jnp.int32)
                return carry
            lax.fori_loop(0, SLOTS // 16, init, 0)

            for j in range(2):
                def scat(c, carry):
                    d = d_v[j, pl.ds(c * 16, 16)]
                    t = lax.iota(jnp.int32, 16) + c * 16
                    plsc.store_scatter(s_v, [d], t, mask=d < SLOTS)
                    return carry
                lax.fori_loop(0, T // 16, scat, 0)

            pltpu.sync_copy(s_v, src_hbm)

    return k(dest2)


ROWS_PER = SLOTS // NW
GCHUNK = 16
NCH = ROWS_PER // GCHUNK


def _dispatch_gather(x, src):
    mesh = plsc.VectorSubcoreMesh(core_axis_name="c", subcore_axis_name="s")

    @functools.partial(
        pl.kernel,
        out_type=jax.ShapeDtypeStruct((SLOTS, H), jnp.float32),
        mesh=mesh,
        scratch_types=[
            pltpu.VMEM((ROWS_PER,), jnp.int32),
            pltpu.VMEM((GCHUNK, H), jnp.float32),
            pltpu.VMEM((GCHUNK, H), jnp.float32),
            pltpu.SemaphoreType.DMA,
            pltpu.SemaphoreType.DMA,
        ],
        compiler_params=pltpu.CompilerParams(needs_layout_passes=False),
    )
    def k(x_hbm, src_hbm, out_hbm, idx_v, buf0, buf1, sem0, sem1):
        wid = lax.axis_index("s") * NC + lax.axis_index("c")
        base = wid * ROWS_PER
        pltpu.sync_copy(src_hbm.at[pl.ds(base, ROWS_PER)], idx_v)
        bufs = (buf0, buf1)
        sems = (sem0, sem1)

        def start_gather(c):
            return pltpu.async_copy(
                x_hbm.at[idx_v.at[pl.ds(c * GCHUNK, GCHUNK)]],
                bufs[c % 2], sems[c % 2])

        cps = [None] * NCH
        cps[0] = start_gather(0)
        for c in range(NCH):
            if c + 1 < NCH:
                cps[c + 1] = start_gather(c + 1)
            cps[c].wait()
            pltpu.sync_copy(bufs[c % 2],
                            out_hbm.at[pl.ds(base + c * GCHUNK, GCHUNK)])

    return k(x, src)


def _ffn_body(dx_ref, wg_ref, wu_ref, wd_ref, out_ref, acc_ref):
    fb = pl.program_id(1)
    a = dx_ref[...]
    g = jnp.dot(a, wg_ref[0], preferred_element_type=jnp.float32)
    u = jnp.dot(a, wu_ref[0], preferred_element_type=jnp.float32)
    hh = g * jax.lax.logistic(g) * u
    contrib = jnp.dot(hh, wd_ref[0], preferred_element_type=jnp.float32)

    @pl.when(fb == 0)
    def _():
        acc_ref[...] = contrib

    @pl.when(fb > 0)
    def _():
        acc_ref[...] += contrib

    @pl.when(fb == NFB - 1)
    def _():
        out_ref[...] = acc_ref[...].astype(jnp.bfloat16)


def _expert_ffn(disp_x, w_gate, w_up, w_down):
    return pl.pallas_call(
        _ffn_body,
        grid=(E, NFB),
        in_specs=[
            pl.BlockSpec((C, H), lambda e, f: (e, 0)),
            pl.BlockSpec((1, H, FB), lambda e, f: (e, 0, f)),
            pl.BlockSpec((1, H, FB), lambda e, f: (e, 0, f)),
            pl.BlockSpec((1, FB, H), lambda e, f: (e, f, 0)),
        ],
        out_specs=pl.BlockSpec((C, H), lambda e, f: (e, 0)),
        out_shape=jax.ShapeDtypeStruct((SLOTS, H), jnp.bfloat16),
        scratch_shapes=[pltpu.VMEM((C, H), jnp.float32)],
        compiler_params=pltpu.CompilerParams(
            vmem_limit_bytes=100 * 1024 * 1024),
    )(disp_x, w_gate, w_up, w_down)


FBS = 256
NFBS = FF // FBS


def _shared_body(x_ref, wg_ref, wu_ref, wd_ref, out_ref):
    fb = pl.program_id(0)
    a = x_ref[...]
    g = jnp.dot(a, wg_ref[...], preferred_element_type=jnp.float32)
    u = jnp.dot(a, wu_ref[...], preferred_element_type=jnp.float32)
    hh = g * jax.lax.logistic(g) * u
    contrib = jnp.dot(hh, wd_ref[...], preferred_element_type=jnp.float32)

    @pl.when(fb == 0)
    def _():
        out_ref[...] = contrib

    @pl.when(fb > 0)
    def _():
        out_ref[...] += contrib


def _shared_ffn(x, ws_gate, ws_up, ws_down):
    return pl.pallas_call(
        _shared_body,
        grid=(NFBS,),
        in_specs=[
            pl.BlockSpec((T, H), lambda f: (0, 0)),
            pl.BlockSpec((H, FBS), lambda f: (0, f)),
            pl.BlockSpec((H, FBS), lambda f: (0, f)),
            pl.BlockSpec((FBS, H), lambda f: (f, 0)),
        ],
        out_specs=pl.BlockSpec((T, H), lambda f: (0, 0)),
        out_shape=jax.ShapeDtypeStruct((T, H), jnp.float32),
        compiler_params=pltpu.CompilerParams(
            vmem_limit_bytes=100 * 1024 * 1024),
    )(x, ws_gate, ws_up, ws_down)


def _combine_body(eo_ref, dest_ref, wts_ref, sh_ref, out_ref):
    sb = pl.program_id(0)
    d0 = dest_ref[...][:, 0:1]
    d1 = dest_ref[...][:, 1:2]
    w0 = wts_ref[...][:, 0:1]
    w1 = wts_ref[...][:, 1:2]
    slot = jax.lax.broadcasted_iota(jnp.int32, (T, C), 1) + sb * C
    D = (jnp.where(d0 == slot, w0, 0.0)
         + jnp.where(d1 == slot, w1, 0.0)).astype(jnp.bfloat16)
    contrib = jnp.dot(D, eo_ref[...], preferred_element_type=jnp.float32)

    @pl.when(sb == 0)
    def _():
        out_ref[...] = sh_ref[...] + contrib

    @pl.when(sb > 0)
    def _():
        out_ref[...] += contrib


def _combine(eo, dest, wts, shared):
    return pl.pallas_call(
        _combine_body,
        grid=(E,),
        in_specs=[
            pl.BlockSpec((C, H), lambda s: (s, 0)),
            pl.BlockSpec((T, 8), lambda s: (0, 0)),
            pl.BlockSpec((T, 8), lambda s: (0, 0)),
            pl.BlockSpec((T, H), lambda s: (0, 0)),
        ],
        out_specs=pl.BlockSpec((T, H), lambda s: (0, 0)),
        out_shape=jax.ShapeDtypeStruct((T, H), jnp.float32),
        compiler_params=pltpu.CompilerParams(
            vmem_limit_bytes=100 * 1024 * 1024),
    )(eo, dest, wts, shared)


def kernel(x, w_router, w_gate, w_up, w_down, ws_gate, ws_up, ws_down):
    dest, wts, aux = _router(x, w_router)
    dest2 = dest.T[:2]
    src = _src_build(dest2)
    disp_x = _dispatch_gather(x, src)
    eo = _expert_ffn(disp_x, w_gate, w_up, w_down)
    shared = _shared_ffn(x, ws_gate, ws_up, ws_down)
    out = _combine(eo, dest, wts, shared)
    return out, aux[0, 0]


# speedup vs baseline: 1.0125x; 1.0125x over previous
import functools

import jax
import jax.numpy as jnp
from jax import lax
from jax.experimental import pallas as pl
from jax.experimental.pallas import tpu as pltpu
from jax.experimental.pallas import tpu_sc as plsc

T = 2048
H = 2048
FF = 5632
E = 8
K = 2
C = 640
SLOTS = E * C
SENT = SLOTS
AUX_COEF = 0.001
Z_COEF = 0.001

FB = 512
NFB = FF // FB

NC = 2
NS = 16
NW = NC * NS


def _cumsum0(a):
    n, e = a.shape
    s = 1
    while s < n:
        a = a + jnp.concatenate([jnp.zeros((s, e), a.dtype), a[:-s]], axis=0)
        s *= 2
    return a


def _router_body(x_ref, wr_ref, dest_ref, wts_ref, aux_ref):
    x = x_ref[...]
    wr = wr_ref[...]
    logits = jnp.dot(x, wr, preferred_element_type=jnp.float32)
    m = jnp.max(logits, axis=-1, keepdims=True)
    ex = jnp.exp(logits - m)
    se = jnp.sum(ex, axis=-1, keepdims=True)
    probs = ex / se
    lse = m + jnp.log(se)

    lane = jax.lax.broadcasted_iota(jnp.int32, (T, E), 1)
    m0 = jnp.max(probs, axis=-1, keepdims=True)
    i0 = jnp.min(jnp.where(probs == m0, lane, E), axis=-1, keepdims=True)
    oh0 = lane == i0
    probs1 = jnp.where(oh0, -jnp.inf, probs)
    m1 = jnp.max(probs1, axis=-1, keepdims=True)
    i1 = jnp.min(jnp.where(probs1 == m1, lane, E), axis=-1, keepdims=True)
    oh1 = lane == i1

    oh0f = oh0.astype(jnp.float32)
    oh1f = oh1.astype(jnp.float32)
    cs0 = _cumsum0(oh0f)
    cnt0 = cs0[T - 1:T, :]
    pos0 = cs0 - oh0f
    cs1 = _cumsum0(oh1f)
    cnt1 = cs1[T - 1:T, :]
    pos1 = cs1 - oh1f + cnt0

    p0 = jnp.sum(pos0 * oh0f, axis=-1, keepdims=True).astype(jnp.int32)
    p1 = jnp.sum(pos1 * oh1f, axis=-1, keepdims=True).astype(jnp.int32)
    keep0 = p0 < C
    keep1 = p1 < C
    sw = m0 + m1
    w0 = jnp.where(keep0, m0 / sw, 0.0)
    w1 = jnp.where(keep1, m1 / sw, 0.0)
    d0 = jnp.where(keep0, i0 * C + p0, SENT)
    d1 = jnp.where(keep1, i1 * C + p1, SENT)

    col = jax.lax.broadcasted_iota(jnp.int32, (T, 8), 1)
    dest_ref[...] = jnp.where(col == 0, d0, jnp.where(col == 1, d1, 0))
    wts_ref[...] = jnp.where(col == 0, w0, jnp.where(col == 1, w1, 0.0))

    f = (cnt0 + cnt1) / T
    p_mean = jnp.sum(probs, axis=0, keepdims=True) / T
    aux = (AUX_COEF * E * jnp.sum(f * p_mean)
           + Z_COEF * jnp.sum(lse * lse) / T)
    aux_ref[...] = jnp.full((1, 1), aux, jnp.float32)


def _router(x, w_router):
    return pl.pallas_call(
        _router_body,
        out_shape=[
            jax.ShapeDtypeStruct((T, 8), jnp.int32),
            jax.ShapeDtypeStruct((T, 8), jnp.float32),
            jax.ShapeDtypeStruct((1, 1), jnp.float32),
        ],
    )(x, w_router)


def _src_build(dest2):
    mesh = plsc.VectorSubcoreMesh(core_axis_name="c", subcore_axis_name="s")

    @functools.partial(
        pl.kernel,
        out_type=jax.ShapeDtypeStruct((SLOTS,), jnp.int32),
        mesh=mesh,
        scratch_types=[
            pltpu.VMEM((2, T), jnp.int32),
            pltpu.VMEM((SLOTS,), jnp.int32),
        ],
        compiler_params=pltpu.CompilerParams(needs_layout_passes=False),
    )
    def k(dest_hbm, src_hbm, d_v, s_v):
        wid = lax.axis_index("s") * NC + lax.axis_index("c")

        @pl.when(wid == 0)
        def _():
            pltpu.sync_copy(dest_hbm, d_v)

            def init(b, carry):
                s_v[pl.ds(b * 16, 16)] = jnp.zeros((16,), jnp.int32)
                return carry
            lax.fori_loop(0, SLOTS // 16, init, 0)

            for j in range(2):
                def scat(c, carry):
                    d = d_v[j, pl.ds(c * 16, 16)]
                    t = lax.iota(jnp.int32, 16) + c * 16
                    plsc.store_scatter(s_v, [d], t, mask=d < SLOTS)
                    return carry
                lax.fori_loop(0, T // 16, scat, 0)

            pltpu.sync_copy(s_v, src_hbm)

    return k(dest2)


ROWS_PER = SLOTS // NW
GCHUNK = 16
NCH = ROWS_PER // GCHUNK


def _dispatch_gather(x, src):
    mesh = plsc.VectorSubcoreMesh(core_axis_name="c", subcore_axis_name="s")

    @functools.partial(
        pl.kernel,
        out_type=jax.ShapeDtypeStruct((SLOTS, H), jnp.float32),
        mesh=mesh,
        scratch_types=[
            pltpu.VMEM((NCH, GCHUNK), jnp.int32),
            pltpu.VMEM((GCHUNK, H), jnp.float32),
            pltpu.VMEM((GCHUNK, H), jnp.float32),
            pltpu.SemaphoreType.DMA,
            pltpu.SemaphoreType.DMA,
        ],
        compiler_params=pltpu.CompilerParams(needs_layout_passes=False),
    )
    def k(x_hbm, src_hbm, out_hbm, idx_v, buf0, buf1, sem0, sem1):
        wid = lax.axis_index("s") * NC + lax.axis_index("c")
        base = wid * ROWS_PER
        for c in range(NCH):
            pltpu.sync_copy(src_hbm.at[pl.ds(base + c * GCHUNK, GCHUNK)],
                            idx_v.at[c])
        bufs = (buf0, buf1)
        sems = (sem0, sem1)
        cps = [None] * NCH
        cps[0] = pltpu.async_copy(x_hbm.at[idx_v.at[0]], bufs[0], sems[0])
        for c in range(NCH):
            if c + 1 < NCH:
                cps[c + 1] = pltpu.async_copy(
                    x_hbm.at[idx_v.at[c + 1]],
                    bufs[(c + 1) % 2], sems[(c + 1) % 2])
            cps[c].wait()
            pltpu.sync_copy(bufs[c % 2],
                            out_hbm.at[pl.ds(base + c * GCHUNK, GCHUNK)])

    return k(x, src)


def _ffn_body(dx_ref, wg_ref, wu_ref, wd_ref, out_ref, acc_ref):
    fb = pl.program_id(1)
    a = dx_ref[...]
    g = jnp.dot(a, wg_ref[0], preferred_element_type=jnp.float32)
    u = jnp.dot(a, wu_ref[0], preferred_element_type=jnp.float32)
    hh = g * jax.lax.logistic(g) * u
    contrib = jnp.dot(hh, wd_ref[0], preferred_element_type=jnp.float32)

    @pl.when(fb == 0)
    def _():
        acc_ref[...] = contrib

    @pl.when(fb > 0)
    def _():
        acc_ref[...] += contrib

    @pl.when(fb == NFB - 1)
    def _():
        out_ref[...] = acc_ref[...].astype(jnp.bfloat16)


def _expert_ffn(disp_x, w_gate, w_up, w_down):
    return pl.pallas_call(
        _ffn_body,
        grid=(E, NFB),
        in_specs=[
            pl.BlockSpec((C, H), lambda e, f: (e, 0)),
            pl.BlockSpec((1, H, FB), lambda e, f: (e, 0, f)),
            pl.BlockSpec((1, H, FB), lambda e, f: (e, 0, f)),
            pl.BlockSpec((1, FB, H), lambda e, f: (e, f, 0)),
        ],
        out_specs=pl.BlockSpec((C, H), lambda e, f: (e, 0)),
        out_shape=jax.ShapeDtypeStruct((SLOTS, H), jnp.bfloat16),
        scratch_shapes=[pltpu.VMEM((C, H), jnp.float32)],
        compiler_params=pltpu.CompilerParams(
            vmem_limit_bytes=100 * 1024 * 1024),
    )(disp_x, w_gate, w_up, w_down)


FBS = 256
NFBS = FF // FBS


def _shared_body(x_ref, wg_ref, wu_ref, wd_ref, out_ref):
    fb = pl.program_id(0)
    a = x_ref[...]
    g = jnp.dot(a, wg_ref[...], preferred_element_type=jnp.float32)
    u = jnp.dot(a, wu_ref[...], preferred_element_type=jnp.float32)
    hh = g * jax.lax.logistic(g) * u
    contrib = jnp.dot(hh, wd_ref[...], preferred_element_type=jnp.float32)

    @pl.when(fb == 0)
    def _():
        out_ref[...] = contrib

    @pl.when(fb > 0)
    def _():
        out_ref[...] += contrib


def _shared_ffn(x, ws_gate, ws_up, ws_down):
    return pl.pallas_call(
        _shared_body,
        grid=(NFBS,),
        in_specs=[
            pl.BlockSpec((T, H), lambda f: (0, 0)),
            pl.BlockSpec((H, FBS), lambda f: (0, f)),
            pl.BlockSpec((H, FBS), lambda f: (0, f)),
            pl.BlockSpec((FBS, H), lambda f: (f, 0)),
        ],
        out_specs=pl.BlockSpec((T, H), lambda f: (0, 0)),
        out_shape=jax.ShapeDtypeStruct((T, H), jnp.float32),
        compiler_params=pltpu.CompilerParams(
            vmem_limit_bytes=100 * 1024 * 1024),
    )(x, ws_gate, ws_up, ws_down)


def _combine_body(eo_ref, dest_ref, wts_ref, sh_ref, out_ref):
    sb = pl.program_id(0)
    d0 = dest_ref[...][:, 0:1]
    d1 = dest_ref[...][:, 1:2]
    w0 = wts_ref[...][:, 0:1]
    w1 = wts_ref[...][:, 1:2]
    slot = jax.lax.broadcasted_iota(jnp.int32, (T, C), 1) + sb * C
    D = (jnp.where(d0 == slot, w0, 0.0)
         + jnp.where(d1 == slot, w1, 0.0)).astype(jnp.bfloat16)
    contrib = jnp.dot(D, eo_ref[...], preferred_element_type=jnp.float32)

    @pl.when(sb == 0)
    def _():
        out_ref[...] = sh_ref[...] + contrib

    @pl.when(sb > 0)
    def _():
        out_ref[...] += contrib


def _combine(eo, dest, wts, shared):
    return pl.pallas_call(
        _combine_body,
        grid=(E,),
        in_specs=[
            pl.BlockSpec((C, H), lambda s: (s, 0)),
            pl.BlockSpec((T, 8), lambda s: (0, 0)),
            pl.BlockSpec((T, 8), lambda s: (0, 0)),
            pl.BlockSpec((T, H), lambda s: (0, 0)),
        ],
        out_specs=pl.BlockSpec((T, H), lambda s: (0, 0)),
        out_shape=jax.ShapeDtypeStruct((T, H), jnp.float32),
        compiler_params=pltpu.CompilerParams(
            vmem_limit_bytes=100 * 1024 * 1024),
    )(eo, dest, wts, shared)


def kernel(x, w_router, w_gate, w_up, w_down, ws_gate, ws_up, ws_down):
    dest, wts, aux = _router(x, w_router)
    dest2 = dest.T[:2]
    src = _src_build(dest2)
    disp_x = _dispatch_gather(x, src)
    eo = _expert_ffn(disp_x, w_gate, w_up, w_down)
    shared = _shared_ffn(x, ws_gate, ws_up, ws_down)
    out = _combine(eo, dest, wts, shared)
    return out, aux[0, 0]


# speedup vs baseline: 1.0156x; 1.0031x over previous
import functools

import jax
import jax.numpy as jnp
from jax import lax
from jax.experimental import pallas as pl
from jax.experimental.pallas import tpu as pltpu
from jax.experimental.pallas import tpu_sc as plsc

T = 2048
H = 2048
FF = 5632
E = 8
K = 2
C = 640
SLOTS = E * C
SENT = SLOTS
AUX_COEF = 0.001
Z_COEF = 0.001

FB = 512
NFB = FF // FB

NC = 2
NS = 16
NW = NC * NS


def _cumsum0(a):
    n, e = a.shape
    s = 1
    while s < n:
        a = a + jnp.concatenate([jnp.zeros((s, e), a.dtype), a[:-s]], axis=0)
        s *= 2
    return a


def _router_body(x_ref, wr_ref, dest_ref, wts_ref, aux_ref):
    x = x_ref[...]
    wr = wr_ref[...]
    logits = jnp.dot(x, wr, preferred_element_type=jnp.float32)
    m = jnp.max(logits, axis=-1, keepdims=True)
    ex = jnp.exp(logits - m)
    se = jnp.sum(ex, axis=-1, keepdims=True)
    probs = ex / se
    lse = m + jnp.log(se)

    lane = jax.lax.broadcasted_iota(jnp.int32, (T, E), 1)
    m0 = jnp.max(probs, axis=-1, keepdims=True)
    i0 = jnp.min(jnp.where(probs == m0, lane, E), axis=-1, keepdims=True)
    oh0 = lane == i0
    probs1 = jnp.where(oh0, -jnp.inf, probs)
    m1 = jnp.max(probs1, axis=-1, keepdims=True)
    i1 = jnp.min(jnp.where(probs1 == m1, lane, E), axis=-1, keepdims=True)
    oh1 = lane == i1

    oh0f = oh0.astype(jnp.float32)
    oh1f = oh1.astype(jnp.float32)
    cs0 = _cumsum0(oh0f)
    cnt0 = cs0[T - 1:T, :]
    pos0 = cs0 - oh0f
    cs1 = _cumsum0(oh1f)
    cnt1 = cs1[T - 1:T, :]
    pos1 = cs1 - oh1f + cnt0

    p0 = jnp.sum(pos0 * oh0f, axis=-1, keepdims=True).astype(jnp.int32)
    p1 = jnp.sum(pos1 * oh1f, axis=-1, keepdims=True).astype(jnp.int32)
    keep0 = p0 < C
    keep1 = p1 < C
    sw = m0 + m1
    w0 = jnp.where(keep0, m0 / sw, 0.0)
    w1 = jnp.where(keep1, m1 / sw, 0.0)
    d0 = jnp.where(keep0, i0 * C + p0, SENT)
    d1 = jnp.where(keep1, i1 * C + p1, SENT)

    col = jax.lax.broadcasted_iota(jnp.int32, (T, 8), 1)
    dest_ref[...] = jnp.where(col == 0, d0, jnp.where(col == 1, d1, 0))
    wts_ref[...] = jnp.where(col == 0, w0, jnp.where(col == 1, w1, 0.0))

    f = (cnt0 + cnt1) / T
    p_mean = jnp.sum(probs, axis=0, keepdims=True) / T
    aux = (AUX_COEF * E * jnp.sum(f * p_mean)
           + Z_COEF * jnp.sum(lse * lse) / T)
    aux_ref[...] = jnp.full((1, 1), aux, jnp.float32)


def _router(x, w_router):
    return pl.pallas_call(
        _router_body,
        out_shape=[
            jax.ShapeDtypeStruct((T, 8), jnp.int32),
            jax.ShapeDtypeStruct((T, 8), jnp.float32),
            jax.ShapeDtypeStruct((1, 1), jnp.float32),
        ],
    )(x, w_router)


def _src_build(dest2):
    mesh = plsc.VectorSubcoreMesh(core_axis_name="c", subcore_axis_name="s")

    @functools.partial(
        pl.kernel,
        out_type=jax.ShapeDtypeStruct((SLOTS,), jnp.int32),
        mesh=mesh,
        scratch_types=[
            pltpu.VMEM((2, T), jnp.int32),
            pltpu.VMEM((SLOTS,), jnp.int32),
        ],
        compiler_params=pltpu.CompilerParams(needs_layout_passes=False),
    )
    def k(dest_hbm, src_hbm, d_v, s_v):
        wid = lax.axis_index("s") * NC + lax.axis_index("c")

        @pl.when(wid == 0)
        def _():
            pltpu.sync_copy(dest_hbm, d_v)

            def init(b, carry):
                s_v[pl.ds(b * 16, 16)] = jnp.zeros((16,), jnp.int32)
                return carry
            lax.fori_loop(0, SLOTS // 16, init, 0)

            for j in range(2):
                def scat(c, carry):
                    d = d_v[j, pl.ds(c * 16, 16)]
                    t = lax.iota(jnp.int32, 16) + c * 16
                    plsc.store_scatter(s_v, [d], t, mask=d < SLOTS)
                    return carry
                lax.fori_loop(0, T // 16, scat, 0)

            pltpu.sync_copy(s_v, src_hbm)

    return k(dest2)


ROWS_PER = SLOTS // NW
GCHUNK = 16
NCH = ROWS_PER // GCHUNK


def _dispatch_gather(x, src):
    mesh = plsc.VectorSubcoreMesh(core_axis_name="c", subcore_axis_name="s")

    @functools.partial(
        pl.kernel,
        out_type=jax.ShapeDtypeStruct((SLOTS, H), jnp.float32),
        mesh=mesh,
        scratch_types=[
            pltpu.VMEM((NCH, GCHUNK), jnp.int32),
            pltpu.VMEM((GCHUNK, H), jnp.float32),
            pltpu.VMEM((GCHUNK, H), jnp.float32),
            pltpu.SemaphoreType.DMA,
            pltpu.SemaphoreType.DMA,
        ],
        compiler_params=pltpu.CompilerParams(needs_layout_passes=False),
    )
    def k(x_hbm, src_hbm, out_hbm, idx_v, buf0, buf1, sem0, sem1):
        wid = lax.axis_index("s") * NC + lax.axis_index("c")
        base = wid * ROWS_PER
        for c in range(NCH):
            pltpu.sync_copy(src_hbm.at[pl.ds(base + c * GCHUNK, GCHUNK)],
                            idx_v.at[c])
        bufs = (buf0, buf1)
        sems = (sem0, sem1)
        cps = [None] * NCH
        cps[0] = pltpu.async_copy(x_hbm.at[idx_v.at[0]], bufs[0], sems[0])
        for c in range(NCH):
            if c + 1 < NCH:
                cps[c + 1] = pltpu.async_copy(
                    x_hbm.at[idx_v.at[c + 1]],
                    bufs[(c + 1) % 2], sems[(c + 1) % 2])
            cps[c].wait()
            pltpu.sync_copy(bufs[c % 2],
                            out_hbm.at[pl.ds(base + c * GCHUNK, GCHUNK)])

    return k(x, src)


def _ffn_body(dx_ref, wg_ref, wu_ref, wd_ref, out_ref, acc_ref):
    fb = pl.program_id(1)
    a = dx_ref[...]
    g = jnp.dot(a, wg_ref[0], preferred_element_type=jnp.float32)
    u = jnp.dot(a, wu_ref[0], preferred_element_type=jnp.float32)
    hh = g * jax.lax.logistic(g) * u
    contrib = jnp.dot(hh, wd_ref[0], preferred_element_type=jnp.float32)

    @pl.when(fb == 0)
    def _():
        acc_ref[...] = contrib

    @pl.when(fb > 0)
    def _():
        acc_ref[...] += contrib

    @pl.when(fb == NFB - 1)
    def _():
        out_ref[...] = acc_ref[...].astype(jnp.bfloat16)


def _expert_ffn(disp_x, w_gate, w_up, w_down):
    return pl.pallas_call(
        _ffn_body,
        grid=(E, NFB),
        in_specs=[
            pl.BlockSpec((C, H), lambda e, f: (e, 0)),
            pl.BlockSpec((1, H, FB), lambda e, f: (e, 0, f)),
            pl.BlockSpec((1, H, FB), lambda e, f: (e, 0, f)),
            pl.BlockSpec((1, FB, H), lambda e, f: (e, f, 0)),
        ],
        out_specs=pl.BlockSpec((C, H), lambda e, f: (e, 0)),
        out_shape=jax.ShapeDtypeStruct((SLOTS, H), jnp.bfloat16),
        scratch_shapes=[pltpu.VMEM((C, H), jnp.float32)],
        compiler_params=pltpu.CompilerParams(
            vmem_limit_bytes=100 * 1024 * 1024),
    )(disp_x, w_gate, w_up, w_down)


FBS = 256
NFBS = FF // FBS


def _shared_body(x_ref, wg_ref, wu_ref, wd_ref, out_ref):
    fb = pl.program_id(0)
    a = x_ref[...]
    g = jnp.dot(a, wg_ref[...], preferred_element_type=jnp.float32)
    u = jnp.dot(a, wu_ref[...], preferred_element_type=jnp.float32)
    hh = g * jax.lax.logistic(g) * u
    contrib = jnp.dot(hh, wd_ref[...], preferred_element_type=jnp.float32)

    @pl.when(fb == 0)
    def _():
        out_ref[...] = contrib

    @pl.when(fb > 0)
    def _():
        out_ref[...] += contrib


def _shared_ffn(x, ws_gate, ws_up, ws_down):
    return pl.pallas_call(
        _shared_body,
        grid=(NFBS,),
        in_specs=[
            pl.BlockSpec((T, H), lambda f: (0, 0)),
            pl.BlockSpec((H, FBS), lambda f: (0, f)),
            pl.BlockSpec((H, FBS), lambda f: (0, f)),
            pl.BlockSpec((FBS, H), lambda f: (f, 0)),
        ],
        out_specs=pl.BlockSpec((T, H), lambda f: (0, 0)),
        out_shape=jax.ShapeDtypeStruct((T, H), jnp.float32),
        compiler_params=pltpu.CompilerParams(
            vmem_limit_bytes=100 * 1024 * 1024),
    )(x, ws_gate, ws_up, ws_down)


def _combine_body(eo_ref, dest_ref, wts_ref, sh_ref, out_ref):
    sb = pl.program_id(0)
    d0 = dest_ref[...][:, 0:1]
    d1 = dest_ref[...][:, 1:2]
    w0 = wts_ref[...][:, 0:1]
    w1 = wts_ref[...][:, 1:2]
    slot = jax.lax.broadcasted_iota(jnp.int32, (T, C), 1) + sb * C
    D = (jnp.where(d0 == slot, w0, 0.0)
         + jnp.where(d1 == slot, w1, 0.0)).astype(jnp.bfloat16)
    contrib = jnp.dot(D, eo_ref[...], preferred_element_type=jnp.float32)

    @pl.when(sb == 0)
    def _():
        out_ref[...] = sh_ref[...] + contrib

    @pl.when(sb > 0)
    def _():
        out_ref[...] += contrib


def _combine(eo, dest, wts, shared):
    return pl.pallas_call(
        _combine_body,
        grid=(E,),
        in_specs=[
            pl.BlockSpec((C, H), lambda s: (s, 0)),
            pl.BlockSpec((T, 8), lambda s: (0, 0)),
            pl.BlockSpec((T, 8), lambda s: (0, 0)),
            pl.BlockSpec((T, H), lambda s: (0, 0)),
        ],
        out_specs=pl.BlockSpec((T, H), lambda s: (0, 0)),
        out_shape=jax.ShapeDtypeStruct((T, H), jnp.float32),
        compiler_params=pltpu.CompilerParams(
            vmem_limit_bytes=100 * 1024 * 1024),
    )(eo, dest, wts, shared)


def kernel(x, w_router, w_gate, w_up, w_down, ws_gate, ws_up, ws_down):
    dest, wts, aux = _router(x, w_router)
    dest2 = dest.T[:2]
    src = _src_build(dest2)
    disp_x = _dispatch_gather(x, src)
    shared = _shared_ffn(x, ws_gate, ws_up, ws_down)
    eo = _expert_ffn(disp_x, w_gate, w_up, w_down)
    out = _combine(eo, dest, wts, shared)
    return out, aux[0, 0]
